# Initial kernel scaffold; baseline (speedup 1.0000x reference)
import jax
import jax.numpy as jnp
from jax.experimental import pallas as pl


def kernel(nodes, edge_feats, senders, receivers, embed_node_W, embed_node_b, embed_edge_W, embed_edge_b, edge_W, edge_b, edge_ls, edge_lb, node_W, node_b, node_ls, node_lb, glob_W, glob_b, glob_ls, glob_lb, dec_W, dec_b):
    raise NotImplementedError("write your pallas kernel here")


# speedup vs baseline: 2.7950x; 2.7950x over previous
import functools

import jax
import jax.numpy as jnp
from jax import lax
from jax.experimental import pallas as pl
from jax.experimental.pallas import tpu as pltpu
from jax.experimental.pallas import tpu_sc as plsc

F32 = jnp.float32
N = 10000
E = 320000
L = 128

NC = 2
NS = 16
NW = NC * NS


def _ln_relu(h, ls, lb):
    mu = jnp.mean(h, axis=-1, keepdims=True)
    d = h - mu
    var = jnp.mean(d * d, axis=-1, keepdims=True)
    return jnp.maximum(d * lax.rsqrt(var + 1e-6) * ls + lb, 0.0)


def _dense(x, w, b, block_rows):
    rows, k = x.shape
    out_cols = w.shape[1]

    def body(x_ref, w_ref, b_ref, o_ref):
        o_ref[...] = jnp.dot(x_ref[...], w_ref[...],
                             preferred_element_type=F32) + b_ref[...]

    return pl.pallas_call(
        body,
        grid=(rows // block_rows,),
        in_specs=[pl.BlockSpec((block_rows, k), lambda i: (i, 0)),
                  pl.BlockSpec((k, out_cols), lambda i: (0, 0)),
                  pl.BlockSpec((1, out_cols), lambda i: (0, 0))],
        out_specs=pl.BlockSpec((block_rows, out_cols), lambda i: (i, 0)),
        out_shape=jax.ShapeDtypeStruct((rows, out_cols), F32),
    )(x, w, b)


def _proj(n, ws, wr, g, wg, eb):
    bn = 2000
    nb = N // bn

    def body(n_ref, ws_ref, wr_ref, g_ref, wg_ref, eb_ref,
             ps_ref, pr_ref, gv_ref):
        x = n_ref[...]
        ps_ref[...] = jnp.dot(x, ws_ref[...], preferred_element_type=F32)
        pr_ref[...] = jnp.dot(x, wr_ref[...], preferred_element_type=F32)

        @pl.when(pl.program_id(0) == 0)
        def _():
            gv_ref[...] = jnp.dot(g_ref[...], wg_ref[...],
                                  preferred_element_type=F32) + eb_ref[...]

    full = lambda i: (0, 0)
    return pl.pallas_call(
        body,
        grid=(nb,),
        in_specs=[pl.BlockSpec((bn, L), lambda i: (i, 0)),
                  pl.BlockSpec((L, L), full),
                  pl.BlockSpec((L, L), full),
                  pl.BlockSpec((1, L), full),
                  pl.BlockSpec((L, L), full),
                  pl.BlockSpec((1, L), full)],
        out_specs=[pl.BlockSpec((bn, L), lambda i: (i, 0)),
                   pl.BlockSpec((bn, L), lambda i: (i, 0)),
                   pl.BlockSpec((1, L), full)],
        out_shape=[jax.ShapeDtypeStruct((N, L), F32),
                   jax.ShapeDtypeStruct((N, L), F32),
                   jax.ShapeDtypeStruct((1, L), F32)],
    )(n, ws, wr, g, wg, eb)


def _edge_mlp(e, s_rows, r_rows, we, gvec, ls, lb):
    be = 4000
    nb = E // be

    def body(e_ref, s_ref, r_ref, we_ref, gv_ref, ls_ref, lb_ref,
             o_ref, agg_ref):
        h = (jnp.dot(e_ref[...], we_ref[...], preferred_element_type=F32)
             + s_ref[...] + r_ref[...] + gv_ref[...])
        a = _ln_relu(h, ls_ref[...], lb_ref[...])
        o_ref[...] = a

        @pl.when(pl.program_id(0) == 0)
        def _():
            agg_ref[...] = jnp.zeros_like(agg_ref)

        agg_ref[...] += jnp.sum(a, axis=0, keepdims=True)

    full = lambda i: (0, 0)
    return pl.pallas_call(
        body,
        grid=(nb,),
        in_specs=[pl.BlockSpec((be, L), lambda i: (i, 0)),
                  pl.BlockSpec((be, L), lambda i: (i, 0)),
                  pl.BlockSpec((be, L), lambda i: (i, 0)),
                  pl.BlockSpec((L, L), full),
                  pl.BlockSpec((1, L), full),
                  pl.BlockSpec((1, L), full),
                  pl.BlockSpec((1, L), full)],
        out_specs=[pl.BlockSpec((be, L), lambda i: (i, 0)),
                   pl.BlockSpec((1, L), full)],
        out_shape=[jax.ShapeDtypeStruct((E, L), F32),
                   jax.ShapeDtypeStruct((1, L), F32)],
    )(e, s_rows, r_rows, we, gvec, ls, lb)


def _node_glob(n, sent, recv, nw, nb_, nls, nlb, g, eagg, gw, gb, gls, glb):
    bn = 2000
    nblk = N // bn

    def body(n_ref, s_ref, r_ref, w_ref, b_ref, ls_ref, lb_ref, g_ref,
             ea_ref, gw_ref, gb_ref, gls_ref, glb_ref,
             o_ref, na_ref, go_ref):
        h = (jnp.dot(n_ref[...], w_ref[0:L, :], preferred_element_type=F32)
             + jnp.dot(s_ref[...], w_ref[L:2 * L, :], preferred_element_type=F32)
             + jnp.dot(r_ref[...], w_ref[2 * L:3 * L, :], preferred_element_type=F32)
             + jnp.dot(g_ref[...], w_ref[3 * L:4 * L, :], preferred_element_type=F32)
             + b_ref[...])
        a = _ln_relu(h, ls_ref[...], lb_ref[...])
        o_ref[...] = a

        @pl.when(pl.program_id(0) == 0)
        def _():
            na_ref[...] = jnp.zeros_like(na_ref)

        na_ref[...] += jnp.sum(a, axis=0, keepdims=True)

        @pl.when(pl.program_id(0) == nblk - 1)
        def _():
            hg = (jnp.dot(na_ref[...], gw_ref[0:L, :], preferred_element_type=F32)
                  + jnp.dot(ea_ref[...], gw_ref[L:2 * L, :], preferred_element_type=F32)
                  + jnp.dot(g_ref[...], gw_ref[2 * L:3 * L, :], preferred_element_type=F32)
                  + gb_ref[...])
            go_ref[...] = _ln_relu(hg, gls_ref[...], glb_ref[...])

    full = lambda i: (0, 0)
    blk = lambda i: (i, 0)
    return pl.pallas_call(
        body,
        grid=(nblk,),
        in_specs=[pl.BlockSpec((bn, L), blk),
                  pl.BlockSpec((bn, L), blk),
                  pl.BlockSpec((bn, L), blk),
                  pl.BlockSpec((4 * L, L), full),
                  pl.BlockSpec((1, L), full),
                  pl.BlockSpec((1, L), full),
                  pl.BlockSpec((1, L), full),
                  pl.BlockSpec((1, L), full),
                  pl.BlockSpec((1, L), full),
                  pl.BlockSpec((3 * L, L), full),
                  pl.BlockSpec((1, L), full),
                  pl.BlockSpec((1, L), full),
                  pl.BlockSpec((1, L), full)],
        out_specs=[pl.BlockSpec((bn, L), blk),
                   pl.BlockSpec((1, L), full),
                   pl.BlockSpec((1, L), full)],
        out_shape=[jax.ShapeDtypeStruct((N, L), F32),
                   jax.ShapeDtypeStruct((1, L), F32),
                   jax.ShapeDtypeStruct((1, L), F32)],
    )(n, sent, recv, nw, nb_, nls, nlb, g, eagg, gw, gb, gls, glb)


_EPW = E // NW
_CG = 80
_EPT = E // NS
_CS = 80

_MESH = plsc.VectorSubcoreMesh(core_axis_name="c", subcore_axis_name="s")


@functools.partial(
    pl.kernel,
    mesh=_MESH,
    out_type=(jax.ShapeDtypeStruct((E, L), F32),
              jax.ShapeDtypeStruct((E, L), F32)),
    scratch_types=[pltpu.VMEM((_CG,), jnp.int32),
                   pltpu.VMEM((_CG,), jnp.int32),
                   pltpu.VMEM((_CG, L), F32),
                   pltpu.VMEM((_CG, L), F32),
                   pltpu.SemaphoreType.DMA,
                   pltpu.SemaphoreType.DMA],
)
def _sc_gather(ps_hbm, pr_hbm, snd_hbm, rcv_hbm, s_out, r_out,
               idx_s, idx_r, rows_s, rows_r, sem_s, sem_r):
    wid = lax.axis_index("s") * NC + lax.axis_index("c")
    base0 = pl.multiple_of(wid * _EPW, 8)

    def chunk(ci, carry):
        base = pl.multiple_of(base0 + ci * _CG, 8)
        pltpu.sync_copy(snd_hbm.at[pl.ds(base, _CG)], idx_s)
        pltpu.sync_copy(rcv_hbm.at[pl.ds(base, _CG)], idx_r)
        cps = pltpu.async_copy(ps_hbm.at[idx_s], rows_s, sem_s)
        cpr = pltpu.async_copy(pr_hbm.at[idx_r], rows_r, sem_r)
        cps.wait()
        cpr.wait()
        pltpu.sync_copy(rows_s, s_out.at[pl.ds(base, _CG)])
        pltpu.sync_copy(rows_r, r_out.at[pl.ds(base, _CG)])
        return carry

    lax.fori_loop(0, _EPW // _CG, chunk, 0)


@functools.partial(
    pl.kernel,
    mesh=_MESH,
    out_type=jax.ShapeDtypeStruct((2, N, L), F32),
    scratch_types=[pltpu.VMEM((_CS,), jnp.int32),
                   pltpu.VMEM((_CS, L), F32),
                   pltpu.VMEM_SHARED((N, L), F32)],
)
def _sc_scatter(e_hbm, idx2_hbm, z_hbm, out_hbm, idx_v, rows_v, acc):
    cid = lax.axis_index("c")
    sid = lax.axis_index("s")

    @pl.when(sid == 0)
    def _():
        pltpu.sync_copy(z_hbm, acc)

    plsc.subcore_barrier()

    base0 = pl.multiple_of(sid * _EPT, 8)
    ibase0 = pl.multiple_of(cid * E + sid * _EPT, 8)

    def chunk(ci, carry):
        base = pl.multiple_of(base0 + ci * _CS, 8)
        ibase = pl.multiple_of(ibase0 + ci * _CS, 8)
        pltpu.sync_copy(idx2_hbm.at[pl.ds(ibase, _CS)], idx_v)
        pltpu.sync_copy(e_hbm.at[pl.ds(base, _CS)], rows_v)
        pltpu.sync_copy(rows_v, acc.at[idx_v], add=True)
        return carry

    lax.fori_loop(0, _EPT // _CS, chunk, 0)
    plsc.subcore_barrier()

    @pl.when(sid == 0)
    def _():
        pltpu.sync_copy(acc, out_hbm.at[cid])


def kernel(nodes, edge_feats, senders, receivers,
           embed_node_W, embed_node_b, embed_edge_W, embed_edge_b,
           edge_W, edge_b, edge_ls, edge_lb,
           node_W, node_b, node_ls, node_lb,
           glob_W, glob_b, glob_ls, glob_lb,
           dec_W, dec_b):
    steps = edge_W.shape[0]
    row = lambda v: v.reshape(1, -1)

    n = _dense(nodes, embed_node_W, row(embed_node_b), 2000)
    e = _dense(edge_feats, embed_edge_W, row(embed_edge_b), 4000)
    g = jnp.zeros((1, L), F32)
    zeros_n = jnp.zeros((N, L), F32)
    snd = senders.astype(jnp.int32)
    rcv = receivers.astype(jnp.int32)
    idx2 = jnp.concatenate([snd, rcv], axis=0)

    for i in range(steps):
        ew = edge_W[i]
        ps, pr, gvec = _proj(n, ew[L:2 * L], ew[2 * L:3 * L],
                             g, ew[3 * L:4 * L], row(edge_b[i]))
        s_rows, r_rows = _sc_gather(ps, pr, snd, rcv)
        e, eagg = _edge_mlp(e, s_rows, r_rows, ew[0:L], gvec,
                            row(edge_ls[i]), row(edge_lb[i]))
        both = _sc_scatter(e, idx2, zeros_n)
        sent, recv = both[0], both[1]
        n, _nagg, g = _node_glob(n, sent, recv, node_W[i], row(node_b[i]),
                                 row(node_ls[i]), row(node_lb[i]),
                                 g, eagg, glob_W[i], row(glob_b[i]),
                                 row(glob_ls[i]), row(glob_lb[i]))

    return _dense(g, dec_W, row(dec_b), 1)


# speedup vs baseline: 4.7974x; 1.7164x over previous
import functools

import jax
import jax.numpy as jnp
from jax import lax
from jax.experimental import pallas as pl
from jax.experimental.pallas import tpu as pltpu
from jax.experimental.pallas import tpu_sc as plsc

F32 = jnp.float32
N = 10000
E = 320000
L = 128

NC = 2
NS = 16
NW = NC * NS


def _ln_relu(h, ls, lb):
    mu = jnp.mean(h, axis=-1, keepdims=True)
    d = h - mu
    var = jnp.mean(d * d, axis=-1, keepdims=True)
    return jnp.maximum(d * lax.rsqrt(var + 1e-6) * ls + lb, 0.0)


def _dense(x, w, b, block_rows):
    rows, k = x.shape
    out_cols = w.shape[1]

    def body(x_ref, w_ref, b_ref, o_ref):
        o_ref[...] = jnp.dot(x_ref[...], w_ref[...],
                             preferred_element_type=F32) + b_ref[...]

    return pl.pallas_call(
        body,
        grid=(rows // block_rows,),
        in_specs=[pl.BlockSpec((block_rows, k), lambda i: (i, 0)),
                  pl.BlockSpec((k, out_cols), lambda i: (0, 0)),
                  pl.BlockSpec((1, out_cols), lambda i: (0, 0))],
        out_specs=pl.BlockSpec((block_rows, out_cols), lambda i: (i, 0)),
        out_shape=jax.ShapeDtypeStruct((rows, out_cols), F32),
    )(x, w, b)


def _proj(n, ws, wr, g, wg, eb):
    bn = 2000
    nb = N // bn

    def body(n_ref, ws_ref, wr_ref, g_ref, wg_ref, eb_ref,
             ps_ref, pr_ref, gv_ref):
        x = n_ref[...]
        ps_ref[...] = jnp.dot(x, ws_ref[...], preferred_element_type=F32)
        pr_ref[...] = jnp.dot(x, wr_ref[...], preferred_element_type=F32)

        @pl.when(pl.program_id(0) == 0)
        def _():
            gv_ref[...] = jnp.dot(g_ref[...], wg_ref[...],
                                  preferred_element_type=F32) + eb_ref[...]

    full = lambda i: (0, 0)
    return pl.pallas_call(
        body,
        grid=(nb,),
        in_specs=[pl.BlockSpec((bn, L), lambda i: (i, 0)),
                  pl.BlockSpec((L, L), full),
                  pl.BlockSpec((L, L), full),
                  pl.BlockSpec((1, L), full),
                  pl.BlockSpec((L, L), full),
                  pl.BlockSpec((1, L), full)],
        out_specs=[pl.BlockSpec((bn, L), lambda i: (i, 0)),
                   pl.BlockSpec((bn, L), lambda i: (i, 0)),
                   pl.BlockSpec((1, L), full)],
        out_shape=[jax.ShapeDtypeStruct((N, L), F32),
                   jax.ShapeDtypeStruct((N, L), F32),
                   jax.ShapeDtypeStruct((1, L), F32)],
    )(n, ws, wr, g, wg, eb)


def _edge_mlp(e, s_rows, r_rows, we, gvec, ls, lb):
    be = 4000
    nb = E // be

    def body(e_ref, s_ref, r_ref, we_ref, gv_ref, ls_ref, lb_ref,
             o_ref, agg_ref):
        h = (jnp.dot(e_ref[...], we_ref[...], preferred_element_type=F32)
             + s_ref[...] + r_ref[...] + gv_ref[...])
        a = _ln_relu(h, ls_ref[...], lb_ref[...])
        o_ref[...] = a

        @pl.when(pl.program_id(0) == 0)
        def _():
            agg_ref[...] = jnp.zeros_like(agg_ref)

        agg_ref[...] += jnp.sum(a, axis=0, keepdims=True)

    full = lambda i: (0, 0)
    return pl.pallas_call(
        body,
        grid=(nb,),
        in_specs=[pl.BlockSpec((be, L), lambda i: (i, 0)),
                  pl.BlockSpec((be, L), lambda i: (i, 0)),
                  pl.BlockSpec((be, L), lambda i: (i, 0)),
                  pl.BlockSpec((L, L), full),
                  pl.BlockSpec((1, L), full),
                  pl.BlockSpec((1, L), full),
                  pl.BlockSpec((1, L), full)],
        out_specs=[pl.BlockSpec((be, L), lambda i: (i, 0)),
                   pl.BlockSpec((1, L), full)],
        out_shape=[jax.ShapeDtypeStruct((E, L), F32),
                   jax.ShapeDtypeStruct((1, L), F32)],
    )(e, s_rows, r_rows, we, gvec, ls, lb)


def _node_glob(n, sent, recv, nw, nb_, nls, nlb, g, eagg, gw, gb, gls, glb):
    bn = 2000
    nblk = N // bn

    def body(n_ref, s_ref, r_ref, w_ref, b_ref, ls_ref, lb_ref, g_ref,
             ea_ref, gw_ref, gb_ref, gls_ref, glb_ref,
             o_ref, na_ref, go_ref):
        h = (jnp.dot(n_ref[...], w_ref[0:L, :], preferred_element_type=F32)
             + jnp.dot(s_ref[...], w_ref[L:2 * L, :], preferred_element_type=F32)
             + jnp.dot(r_ref[...], w_ref[2 * L:3 * L, :], preferred_element_type=F32)
             + jnp.dot(g_ref[...], w_ref[3 * L:4 * L, :], preferred_element_type=F32)
             + b_ref[...])
        a = _ln_relu(h, ls_ref[...], lb_ref[...])
        o_ref[...] = a

        @pl.when(pl.program_id(0) == 0)
        def _():
            na_ref[...] = jnp.zeros_like(na_ref)

        na_ref[...] += jnp.sum(a, axis=0, keepdims=True)

        @pl.when(pl.program_id(0) == nblk - 1)
        def _():
            hg = (jnp.dot(na_ref[...], gw_ref[0:L, :], preferred_element_type=F32)
                  + jnp.dot(ea_ref[...], gw_ref[L:2 * L, :], preferred_element_type=F32)
                  + jnp.dot(g_ref[...], gw_ref[2 * L:3 * L, :], preferred_element_type=F32)
                  + gb_ref[...])
            go_ref[...] = _ln_relu(hg, gls_ref[...], glb_ref[...])

    full = lambda i: (0, 0)
    blk = lambda i: (i, 0)
    return pl.pallas_call(
        body,
        grid=(nblk,),
        in_specs=[pl.BlockSpec((bn, L), blk),
                  pl.BlockSpec((bn, L), blk),
                  pl.BlockSpec((bn, L), blk),
                  pl.BlockSpec((4 * L, L), full),
                  pl.BlockSpec((1, L), full),
                  pl.BlockSpec((1, L), full),
                  pl.BlockSpec((1, L), full),
                  pl.BlockSpec((1, L), full),
                  pl.BlockSpec((1, L), full),
                  pl.BlockSpec((3 * L, L), full),
                  pl.BlockSpec((1, L), full),
                  pl.BlockSpec((1, L), full),
                  pl.BlockSpec((1, L), full)],
        out_specs=[pl.BlockSpec((bn, L), blk),
                   pl.BlockSpec((1, L), full),
                   pl.BlockSpec((1, L), full)],
        out_shape=[jax.ShapeDtypeStruct((N, L), F32),
                   jax.ShapeDtypeStruct((1, L), F32),
                   jax.ShapeDtypeStruct((1, L), F32)],
    )(n, sent, recv, nw, nb_, nls, nlb, g, eagg, gw, gb, gls, glb)


_EPW = E // NW
_CG = 80
_EPT = E // NS
_CS = 40

_MESH = plsc.VectorSubcoreMesh(core_axis_name="c", subcore_axis_name="s")

_NBG = 5
_NGRP = _EPW // _CG // _NBG


@functools.partial(
    pl.kernel,
    mesh=_MESH,
    out_type=(jax.ShapeDtypeStruct((E, L), F32),
              jax.ShapeDtypeStruct((E, L), F32)),
    scratch_types=[pltpu.VMEM((_EPW,), jnp.int32),
                   pltpu.VMEM((_EPW,), jnp.int32),
                   pltpu.VMEM((_NBG, _CG, L), F32),
                   pltpu.VMEM((_NBG, _CG, L), F32),
                   pltpu.SemaphoreType.DMA((_NBG,)),
                   pltpu.SemaphoreType.DMA((_NBG,)),
                   pltpu.SemaphoreType.DMA((_NBG,)),
                   pltpu.SemaphoreType.DMA((_NBG,))],
)
def _sc_gather(ps_hbm, pr_hbm, snd_hbm, rcv_hbm, s_out, r_out,
               idx_s, idx_r, rows_s, rows_r,
               sem_gs, sem_gr, sem_ws, sem_wr):
    wid = lax.axis_index("s") * NC + lax.axis_index("c")
    base0 = pl.multiple_of(wid * _EPW, 8)

    pltpu.sync_copy(snd_hbm.at[pl.ds(base0, _EPW)], idx_s)
    pltpu.sync_copy(rcv_hbm.at[pl.ds(base0, _EPW)], idx_r)

    def start_gather(b, ci):
        off = pl.multiple_of(ci * _CG, 8)
        pltpu.async_copy(ps_hbm.at[idx_s.at[pl.ds(off, _CG)]],
                         rows_s.at[b], sem_gs.at[b])
        pltpu.async_copy(pr_hbm.at[idx_r.at[pl.ds(off, _CG)]],
                         rows_r.at[b], sem_gr.at[b])

    for b in range(_NBG):
        start_gather(b, b)

    def group(g, carry):
        for b in range(_NBG):
            ci = g * _NBG + b
            base = pl.multiple_of(base0 + ci * _CG, 8)
            pltpu.make_async_copy(ps_hbm.at[idx_s.at[pl.ds(0, _CG)]],
                                  rows_s.at[b], sem_gs.at[b]).wait()
            pltpu.make_async_copy(pr_hbm.at[idx_r.at[pl.ds(0, _CG)]],
                                  rows_r.at[b], sem_gr.at[b]).wait()
            ws = pltpu.async_copy(rows_s.at[b], s_out.at[pl.ds(base, _CG)],
                                  sem_ws.at[b])
            wr = pltpu.async_copy(rows_r.at[b], r_out.at[pl.ds(base, _CG)],
                                  sem_wr.at[b])
            ws.wait()
            wr.wait()

            @pl.when(g < _NGRP - 1)
            def _():
                start_gather(b, ci + _NBG)
        return carry

    lax.fori_loop(0, _NGRP, group, 0)


_NBS = 5
_NGRPS = _EPT // _CS // _NBS


@functools.partial(
    pl.kernel,
    mesh=_MESH,
    out_type=jax.ShapeDtypeStruct((2, N, L), F32),
    scratch_types=[pltpu.VMEM((_NBS, _CS), jnp.int32),
                   pltpu.VMEM((_NBS, _CS, L), F32),
                   pltpu.VMEM_SHARED((N, L), F32),
                   pltpu.SemaphoreType.DMA((_NBS,)),
                   pltpu.SemaphoreType.DMA((_NBS,))],
)
def _sc_scatter(e_hbm, idx2_hbm, z_hbm, out_hbm,
                idx_b, rows_v, acc, sem_ld, sem_ix):
    cid = lax.axis_index("c")
    sid = lax.axis_index("s")

    @pl.when(sid == 0)
    def _():
        pltpu.sync_copy(z_hbm, acc)

    base0 = pl.multiple_of(sid * _EPT, 8)
    ibase0 = pl.multiple_of(cid * E + sid * _EPT, 8)
    plsc.subcore_barrier()

    def start_load(b, ci):
        base = pl.multiple_of(base0 + ci * _CS, 8)
        ibase = pl.multiple_of(ibase0 + ci * _CS, 8)
        pltpu.async_copy(idx2_hbm.at[pl.ds(ibase, _CS)], idx_b.at[b],
                         sem_ix.at[b])
        pltpu.async_copy(e_hbm.at[pl.ds(base, _CS)], rows_v.at[b],
                         sem_ld.at[b])

    for b in range(_NBS):
        start_load(b, b)

    def group(g, carry):
        for b in range(_NBS):
            ci = g * _NBS + b
            pltpu.make_async_copy(idx2_hbm.at[pl.ds(0, _CS)], idx_b.at[b],
                                  sem_ix.at[b]).wait()
            pltpu.make_async_copy(e_hbm.at[pl.ds(0, _CS)], rows_v.at[b],
                                  sem_ld.at[b]).wait()
            pltpu.sync_copy(rows_v.at[b], acc.at[idx_b.at[b]], add=True)

            @pl.when(g < _NGRPS - 1)
            def _():
                start_load(b, ci + _NBS)
        return carry

    lax.fori_loop(0, _NGRPS, group, 0)
    plsc.subcore_barrier()

    @pl.when(sid == 0)
    def _():
        pltpu.sync_copy(acc, out_hbm.at[cid])


def kernel(nodes, edge_feats, senders, receivers,
           embed_node_W, embed_node_b, embed_edge_W, embed_edge_b,
           edge_W, edge_b, edge_ls, edge_lb,
           node_W, node_b, node_ls, node_lb,
           glob_W, glob_b, glob_ls, glob_lb,
           dec_W, dec_b):
    steps = edge_W.shape[0]
    row = lambda v: v.reshape(1, -1)

    n = _dense(nodes, embed_node_W, row(embed_node_b), 2000)
    e = _dense(edge_feats, embed_edge_W, row(embed_edge_b), 4000)
    g = jnp.zeros((1, L), F32)
    zeros_n = jnp.zeros((N, L), F32)
    snd = senders.astype(jnp.int32)
    rcv = receivers.astype(jnp.int32)
    idx2 = jnp.concatenate([snd, rcv], axis=0)

    for i in range(steps):
        ew = edge_W[i]
        ps, pr, gvec = _proj(n, ew[L:2 * L], ew[2 * L:3 * L],
                             g, ew[3 * L:4 * L], row(edge_b[i]))
        s_rows, r_rows = _sc_gather(ps, pr, snd, rcv)
        e, eagg = _edge_mlp(e, s_rows, r_rows, ew[0:L], gvec,
                            row(edge_ls[i]), row(edge_lb[i]))
        both = _sc_scatter(e, idx2, zeros_n)
        sent, recv = both[0], both[1]
        n, _nagg, g = _node_glob(n, sent, recv, node_W[i], row(node_b[i]),
                                 row(node_ls[i]), row(node_lb[i]),
                                 g, eagg, glob_W[i], row(glob_b[i]),
                                 row(glob_ls[i]), row(glob_lb[i]))

    return _dense(g, dec_W, row(dec_b), 1)
